# baseline (speedup 1.0000x reference)
import functools
import jax
import jax.numpy as jnp
from jax import lax
from jax.experimental import pallas as pl
from jax.experimental.pallas import tpu as pltpu
from jax.experimental.pallas import tpu_sc as plsc

NX = 768
NF = 768
NX8 = NX // 8
NBITS = 8

NC, NS = 2, 16
NW = NC * NS
FPW = NF // NW
GV = NX8 // 16

_M1 = 0x01010101
_MT = 0x08040201


def _sc_decode_body(codes_hbm, scale_hbm, wq_hbm,
                    codes_v, scale_v, out_v, slut_v):
    wid = lax.axis_index("s") * NC + lax.axis_index("c")
    f_base = wid * FPW
    pltpu.sync_copy(scale_hbm.at[pl.ds(f_base, FPW)], scale_v)
    pltpu.sync_copy(codes_hbm.at[:, pl.ds(f_base, FPW), :], codes_v)

    iota = lax.broadcasted_iota(jnp.int32, (16,), 0)
    iota8 = iota * 8

    def fl_body(fl, carry):
        f_abs = f_base + fl
        svvec = scale_v[fl, :]
        sv = [jnp.full((16,), svvec[i], jnp.float32) for i in range(NBITS)]
        lo = jnp.zeros((16,), jnp.float32)
        hi = jnp.zeros((16,), jnp.float32)
        for j in range(4):
            bit = (iota >> j) & 1
            lo = lo + jnp.where(bit != 0, sv[j], -sv[j])
            hi = hi + jnp.where(bit != 0, sv[4 + j], -sv[4 + j])
        for k in range(16):
            slut_v[pl.ds(k * 16, 16)] = lo + hi[k]

        for gv in range(GV):
            gsl = pl.ds(gv * 16, 16)
            v = [codes_v[i, fl, gsl] for i in range(NBITS)]
            pack_a = (v[0] << 24) | (v[1] << 16) | (v[2] << 8) | v[3]
            pack_b = (v[4] << 24) | (v[5] << 16) | (v[6] << 8) | v[7]
            for p in range(8):
                a = (pack_a >> (7 - p)) & _M1
                pa = ((a * _MT) >> 24) & 0xF
                b = (pack_b >> (7 - p)) & _M1
                pb = ((b * _MT) >> 24) & 0xF
                patt = pa | (pb << 4)
                val = plsc.load_gather(slut_v, [patt])
                cidx = iota8 + (128 * gv + p)
                plsc.store_scatter(out_v, [jnp.full((16,), fl, jnp.int32), cidx], val)
        return carry

    lax.fori_loop(0, FPW, fl_body, 0, unroll=2)
    pltpu.sync_copy(out_v, wq_hbm.at[pl.ds(f_base, FPW)])


def _tc_matmul_body(x_ref, wq_ref, bias_ref, out_ref):
    out = lax.dot_general(
        x_ref[...], wq_ref[...], (((1,), (1,)), ((), ())),
        preferred_element_type=jnp.float32,
    )
    out_ref[...] = out + bias_ref[...]


def kernel(x, scale, bias, binary):
    size_out = x.shape[:-1] + (NF,)
    x2 = x.reshape(-1, NX)
    scale_pad = jnp.concatenate(
        [scale.reshape(NBITS, NF).T,
         jnp.zeros((NF, 16 - NBITS), jnp.float32)], axis=1)

    sc_decode = functools.partial(
        pl.kernel,
        out_type=jax.ShapeDtypeStruct((NF, NX), jnp.float32),
        mesh=plsc.VectorSubcoreMesh(
            core_axis_name="c", subcore_axis_name="s",
            num_cores=NC, num_subcores=NS,
        ),
        compiler_params=pltpu.CompilerParams(
            needs_layout_passes=False, use_tc_tiling_on_sc=True),
        scratch_types=[
            pltpu.VMEM((NBITS, FPW, NX8), jnp.int32),
            pltpu.VMEM((FPW, 16), jnp.float32),
            pltpu.VMEM((FPW, NX), jnp.float32),
            pltpu.VMEM((256,), jnp.float32),
        ],
    )(_sc_decode_body)
    wq = sc_decode(binary, scale_pad)

    out = pl.pallas_call(
        _tc_matmul_body,
        out_shape=jax.ShapeDtypeStruct((x2.shape[0], NF), jnp.float32),
    )(x2, wq, bias.reshape(1, NF))
    return out.reshape(size_out)


# speedup vs baseline: 1.9145x; 1.9145x over previous
import jax
import jax.numpy as jnp
from jax import lax
from jax.experimental import pallas as pl
from jax.experimental.pallas import tpu as pltpu

NX = 768
NF = 768
NX8 = NX // 8
NBITS = 8


def _body(x_ref, scale_ref, bias_ref, binary_ref, out_ref):
    g_row = lax.broadcasted_iota(jnp.int32, (NX8, NX), 0)
    c_col = lax.broadcasted_iota(jnp.int32, (NX8, NX), 1)
    expand = jnp.where(c_col // 8 == g_row, 1.0, 0.0).astype(jnp.float32)

    col = lax.broadcasted_iota(jnp.int32, (NF, NX), 1)
    shift = 7 - (col % 8)

    wq = jnp.zeros((NF, NX), jnp.float32)
    for i in range(NBITS):
        codes = binary_ref[i].astype(jnp.float32)
        codes_exp = lax.dot_general(
            codes, expand, (((1,), (0,)), ((), ())),
            preferred_element_type=jnp.float32,
        )
        bits = (codes_exp.astype(jnp.int32) >> shift) & 1
        signs = (2 * bits - 1).astype(jnp.float32)
        wq = wq + scale_ref[i] * signs

    out = lax.dot_general(
        x_ref[...], wq, (((1,), (1,)), ((), ())),
        preferred_element_type=jnp.float32,
    )
    out_ref[...] = out + bias_ref[...]


def kernel(x, scale, bias, binary):
    size_out = x.shape[:-1] + (NF,)
    x2 = x.reshape(-1, NX)
    out = pl.pallas_call(
        _body,
        out_shape=jax.ShapeDtypeStruct((x2.shape[0], NF), jnp.float32),
    )(x2, scale, bias.reshape(1, NF), binary)
    return out.reshape(size_out)
